# Initial kernel scaffold; baseline (speedup 1.0000x reference)
import jax
import jax.numpy as jnp
from jax.experimental import pallas as pl


def kernel(feat, scores_rgb, scores_flow):
    raise NotImplementedError("write your pallas kernel here")


# speedup vs baseline: 2.7726x; 2.7726x over previous
import jax
import jax.numpy as jnp
from jax import lax
from jax.experimental import pallas as pl
from jax.experimental.pallas import tpu as pltpu
from jax.experimental.pallas import tpu_sc as plsc

B, T, F = 32, 8192, 256
L = 16
NCH = T // L
UNROLL = 8

_NEG = float("-inf")
_POS = float("inf")


def _butterfly(x, tmp_ref, combine, lane):
  for s in (8, 4, 2, 1):
    tmp_ref[...] = x
    x = combine(x, plsc.load_gather(tmp_ref, [lane ^ s]))
  return x


def _topk(key_ref, k, lane, g, base, tf, ti, produce=None, src_ref=None):
  for j in range(k):
    first = j == 0 and produce is not None

    def sweep(i, carry, first=first):
      bv, bi, ci = carry
      for u in range(UNROLL):
        sl = pl.ds((i * UNROLL + u) * L, L)
        if first:
          x = produce(src_ref[sl])
          key_ref[sl] = x
        else:
          x = key_ref[sl]
        m = x > bv
        bv = jnp.maximum(bv, x)
        bi = jnp.where(m, ci, bi)
        ci = ci + L
      return bv, bi, ci

    bv0 = jnp.full((L,), _NEG, jnp.float32)
    bv, bi, _ = lax.fori_loop(0, NCH // UNROLL, sweep,
                              (bv0, jnp.zeros((L,), jnp.int32), lane))
    mx = _butterfly(bv, tf, jnp.maximum, lane)
    gi = _butterfly(jnp.where(bv == mx, bi, T), ti, jnp.minimum, lane)
    g = jnp.where(lane == base + j, gi, g)
    plsc.store_scatter(key_ref, [gi], jnp.full((L,), _NEG, jnp.float32),
                       mask=lane == 0)
  return g


def _body(feat_hbm, srgb_hbm, sflow_hbm, out_rows,
          s_v, f_v, k_v, tf, ti, h_ref, idx_v, rows_v, sem):
  lane = lax.iota(jnp.int32, L)
  wid = lax.axis_index("c") * 16 + lax.axis_index("s")

  pltpu.sync_copy(srgb_hbm.at[wid], s_v)
  pltpu.sync_copy(sflow_hbm.at[wid], f_v)

  h = _topk(k_v, 10, lane, jnp.zeros((L,), jnp.int32), 0, tf, ti,
            produce=lambda x: -jnp.abs(x - 0.5), src_ref=s_v)
  h_ref[...] = h
  flow_h = plsc.load_gather(f_v, [h])

  g = jnp.zeros((L,), jnp.int32)

  v = jnp.where(lane < 10, flow_h, _NEG)
  for j in range(3):
    mx = _butterfly(v, tf, jnp.maximum, lane)
    p = _butterfly(jnp.where(v == mx, lane, L), ti, jnp.minimum, lane)
    t = plsc.load_gather(h_ref, [p])
    g = jnp.where(lane == 1 + j, t, g)
    v = jnp.where(lane == p, _NEG, v)

  v2 = jnp.where(lane < 10, flow_h, _POS)
  mn = _butterfly(v2, tf, jnp.minimum, lane)
  p2 = _butterfly(jnp.where(v2 == mn, lane, L), ti, jnp.minimum, lane)
  g = jnp.where(lane == 0, plsc.load_gather(h_ref, [p2]), g)

  g = _topk(k_v, 2, lane, g, 4, tf, ti, produce=lambda x: -x, src_ref=s_v)

  g = _topk(s_v, 10, lane, g, 6, tf, ti)

  idx_v[...] = g + wid * T
  pltpu.async_copy(feat_hbm.at[idx_v], rows_v, sem).wait()
  pltpu.sync_copy(rows_v, out_rows.at[wid])


@jax.jit
def kernel(feat, scores_rgb, scores_flow):
  feat_flat = feat.reshape(B * T, F)
  f32 = jnp.float32
  run = pl.kernel(
      _body,
      out_type=jax.ShapeDtypeStruct((B, L, F), f32),
      mesh=plsc.VectorSubcoreMesh(core_axis_name="c", subcore_axis_name="s",
                                  num_cores=2, num_subcores=16),
      compiler_params=pltpu.CompilerParams(needs_layout_passes=False),
      scratch_types=[
          pltpu.VMEM((T,), f32),
          pltpu.VMEM((T,), f32),
          pltpu.VMEM((T,), f32),
          pltpu.VMEM((L,), f32),
          pltpu.VMEM((L,), jnp.int32),
          pltpu.VMEM((L,), jnp.int32),
          pltpu.VMEM((L,), jnp.int32),
          pltpu.VMEM((L, F), f32),
          pltpu.SemaphoreType.DMA,
      ],
  )
  rows = run(feat_flat, scores_rgb, scores_flow)
  return (rows[:, 0:1], rows[:, 1:4], rows[:, 4:6], rows[:, 6:16])
